# baseline (speedup 1.0000x reference)
import functools

import jax
import jax.numpy as jnp
from jax import lax
from jax.experimental import pallas as pl
from jax.experimental.pallas import tpu as pltpu
from jax.experimental.pallas import tpu_sc as plsc

_H, _W = 320, 800
_THR = 0.1
_NW = 32
_RPW = _H // _NW
_CW = _RPW * _W
_OW = _CW * 5
_NCHUNK = _CW // 16

_mesh = plsc.VectorSubcoreMesh(core_axis_name="c", subcore_axis_name="s")


@functools.partial(
    pl.kernel,
    out_type=jax.ShapeDtypeStruct((_H * _W * 5,), jnp.float32),
    mesh=_mesh,
    compiler_params=pltpu.CompilerParams(
        needs_layout_passes=False, skip_device_barrier=True
    ),
    scratch_types=[
        pltpu.VMEM((_CW + 32,), jnp.float32),
        pltpu.VMEM((_CW,), jnp.float32),
        pltpu.VMEM((_CW,), jnp.float32),
        pltpu.VMEM((_CW,), jnp.float32),
        pltpu.VMEM((_CW,), jnp.float32),
        pltpu.VMEM((_OW,), jnp.float32),
    ],
)
def _decode(heat_hbm, offx_hbm, offy_hbm, errx_hbm, erry_hbm, out_hbm,
            hbuf, oxbuf, oybuf, exbuf, eybuf, obuf):
    cid = lax.axis_index("c")
    sid = lax.axis_index("s")
    wid = cid * 16 + sid
    base = wid * _CW

    pltpu.sync_copy(heat_hbm.at[pl.ds(base, _CW)], hbuf.at[pl.ds(16, _CW)])
    pltpu.sync_copy(offx_hbm.at[pl.ds(base, _CW)], oxbuf)
    pltpu.sync_copy(offy_hbm.at[pl.ds(base, _CW)], oybuf)
    pltpu.sync_copy(errx_hbm.at[pl.ds(base, _CW)], exbuf)
    pltpu.sync_copy(erry_hbm.at[pl.ds(base, _CW)], eybuf)

    lanes = lax.iota(jnp.int32, 16)
    lanes_f = lanes.astype(jnp.float32)
    idx5 = lanes * 5
    ninf = jnp.full((16,), -jnp.inf, dtype=jnp.float32)
    zero = jnp.zeros((16,), dtype=jnp.float32)
    y0 = wid * _RPW

    def chunk(i, carry):
        w0 = i * 16
        x0 = w0 % _W
        cen = hbuf[pl.ds(w0 + 16, 16)]
        lft = hbuf[pl.ds(w0 + 15, 16)]
        rgt = hbuf[pl.ds(w0 + 17, 16)]
        xs = x0 + lanes
        lft = jnp.where(xs == 0, ninf, lft)
        rgt = jnp.where(xs == _W - 1, ninf, rgt)
        hmax = jnp.maximum(jnp.maximum(lft, rgt), cen)
        nms = jnp.where(hmax == cen, cen, zero)
        m = nms > _THR
        ox = oxbuf[pl.ds(w0, 16)]
        oy = oybuf[pl.ds(w0, 16)]
        ex = exbuf[pl.ds(w0, 16)]
        ey = eybuf[pl.ds(w0, 16)]
        xf = x0.astype(jnp.float32) + lanes_f
        yf = (y0 + w0 // _W).astype(jnp.float32)
        rootx = jnp.where(m, xf + ox, zero)
        rooty = jnp.where(m, yf + oy, zero)
        alignx = jnp.where(m, xf + ex, zero)
        aligny = jnp.where(m, yf + ey, zero)
        ob = idx5 + w0 * 5
        plsc.store_scatter(obuf, [ob], nms)
        plsc.store_scatter(obuf, [ob + 1], rootx)
        plsc.store_scatter(obuf, [ob + 2], rooty)
        plsc.store_scatter(obuf, [ob + 3], alignx)
        plsc.store_scatter(obuf, [ob + 4], aligny)
        return carry

    lax.fori_loop(0, _NCHUNK, chunk, 0)
    pltpu.sync_copy(obuf, out_hbm.at[pl.ds(base * 5, _OW)])


def kernel(heat, offset, error):
    hf = heat.reshape(_H * _W)
    off = offset.reshape(2, _H * _W)
    err = error.reshape(2, _H * _W)
    out = _decode(hf, off[0], off[1], err[0], err[1])
    return out.reshape(_H, _W, 5)


# speedup vs baseline: 17.6334x; 17.6334x over previous
import jax
import jax.numpy as jnp
from jax.experimental import pallas as pl

_H, _W = 320, 800
_THR = 0.1
_HB = 16


def _decode_body(heat_ref, off_ref, err_ref, out_ref):
    h = heat_ref[...]
    ninf = jnp.full((_HB, 1), -jnp.inf, dtype=jnp.float32)
    lft = jnp.concatenate([ninf, h[:, :-1]], axis=1)
    rgt = jnp.concatenate([h[:, 1:], ninf], axis=1)
    hmax = jnp.maximum(jnp.maximum(lft, rgt), h)
    nms = jnp.where(hmax == h, h, 0.0)
    m = nms > _THR

    xs = jax.lax.broadcasted_iota(jnp.int32, (_HB, _W), 1).astype(jnp.float32)
    ys = (pl.program_id(0) * _HB).astype(jnp.float32) + jax.lax.broadcasted_iota(
        jnp.int32, (_HB, _W), 0
    ).astype(jnp.float32)

    out_ref[0] = nms
    out_ref[1] = jnp.where(m, xs + off_ref[0], 0.0)
    out_ref[2] = jnp.where(m, ys + off_ref[1], 0.0)
    out_ref[3] = jnp.where(m, xs + err_ref[0], 0.0)
    out_ref[4] = jnp.where(m, ys + err_ref[1], 0.0)


@jax.jit
def _decode(heat2d, off, err):
    return pl.pallas_call(
        _decode_body,
        grid=(_H // _HB,),
        in_specs=[
            pl.BlockSpec((_HB, _W), lambda i: (i, 0)),
            pl.BlockSpec((2, _HB, _W), lambda i: (0, i, 0)),
            pl.BlockSpec((2, _HB, _W), lambda i: (0, i, 0)),
        ],
        out_specs=pl.BlockSpec((5, _HB, _W), lambda i: (0, i, 0)),
        out_shape=jax.ShapeDtypeStruct((5, _H, _W), jnp.float32),
    )(heat2d, off, err)


def kernel(heat, offset, error):
    hf = heat.reshape(_H, _W)
    off = offset.reshape(2, _H, _W)
    err = error.reshape(2, _H, _W)
    out5 = _decode(hf, off, err)
    return jnp.transpose(out5, (1, 2, 0))


# speedup vs baseline: 34.0573x; 1.9314x over previous
import jax
import jax.numpy as jnp
from jax.experimental import pallas as pl

_H, _W = 320, 800
_THR = 0.1
_HB = 64


def _decode_body(heat_ref, off_ref, err_ref, out_ref):
    h = heat_ref[...]
    ninf = jnp.full((_HB, 1), -jnp.inf, dtype=jnp.float32)
    lft = jnp.concatenate([ninf, h[:, :-1]], axis=1)
    rgt = jnp.concatenate([h[:, 1:], ninf], axis=1)
    hmax = jnp.maximum(jnp.maximum(lft, rgt), h)
    nms = jnp.where(hmax == h, h, 0.0)
    m = nms > _THR

    xs = jax.lax.broadcasted_iota(jnp.int32, (_HB, _W), 1).astype(jnp.float32)
    ys = (pl.program_id(0) * _HB).astype(jnp.float32) + jax.lax.broadcasted_iota(
        jnp.int32, (_HB, _W), 0
    ).astype(jnp.float32)

    out_ref[0] = nms
    out_ref[1] = jnp.where(m, xs + off_ref[0], 0.0)
    out_ref[2] = jnp.where(m, ys + off_ref[1], 0.0)
    out_ref[3] = jnp.where(m, xs + err_ref[0], 0.0)
    out_ref[4] = jnp.where(m, ys + err_ref[1], 0.0)


@jax.jit
def _decode(heat2d, off, err):
    return pl.pallas_call(
        _decode_body,
        grid=(_H // _HB,),
        in_specs=[
            pl.BlockSpec((_HB, _W), lambda i: (i, 0)),
            pl.BlockSpec((2, _HB, _W), lambda i: (0, i, 0)),
            pl.BlockSpec((2, _HB, _W), lambda i: (0, i, 0)),
        ],
        out_specs=pl.BlockSpec((5, _HB, _W), lambda i: (0, i, 0)),
        out_shape=jax.ShapeDtypeStruct((5, _H, _W), jnp.float32),
    )(heat2d, off, err)


def kernel(heat, offset, error):
    hf = heat.reshape(_H, _W)
    off = offset.reshape(2, _H, _W)
    err = error.reshape(2, _H, _W)
    out5 = _decode(hf, off, err)
    return jnp.transpose(out5, (1, 2, 0))
